# baseline (speedup 1.0000x reference)
import functools

import jax
import jax.numpy as jnp
from jax import lax
from jax.experimental import pallas as pl
from jax.experimental.pallas import tpu as pltpu
from jax.experimental.pallas import tpu_sc as plsc

VOCAB = 1000000
DIM = 64
B = 4096
L = 200

NC = 2
NS = 16
NW = NC * NS
BPW = B // NW
NCHUNK = 2
CH = L // NCHUNK
LANES = 16
NREG = DIM // LANES


def _build_sc_kernel():
  mesh = plsc.VectorSubcoreMesh(core_axis_name="c", subcore_axis_name="s")

  @functools.partial(
      pl.kernel,
      mesh=mesh,
      compiler_params=pltpu.CompilerParams(use_tc_tiling_on_sc=False),
      out_type=jax.ShapeDtypeStruct((B, DIM), jnp.float32),
      scratch_types=[
          pltpu.VMEM((BPW, NCHUNK, CH), jnp.int32),
          pltpu.VMEM((BPW,), jnp.float32),
          pltpu.VMEM((L, DIM), jnp.float32),
          pltpu.VMEM((BPW, DIM), jnp.float32),
          pltpu.SemaphoreType.DMA,
      ],
  )
  def k(q_hbm, len_hbm, w_hbm, out_hbm, idx_v, len_v, rows_v, out_v, sem):
    wid = lax.axis_index("s") * NC + lax.axis_index("c")
    base = wid * BPW

    pltpu.sync_copy(q_hbm.at[pl.ds(base, BPW)], idx_v)
    pltpu.sync_copy(len_hbm.at[pl.ds(base, BPW)], len_v)

    def item_body(b, _):
      cp0 = pltpu.async_copy(
          w_hbm.at[idx_v.at[b, 0]], rows_v.at[pl.ds(0, CH)], sem)
      cp1 = pltpu.async_copy(
          w_hbm.at[idx_v.at[b, 1]], rows_v.at[pl.ds(CH, CH)], sem)
      cp0.wait()
      cp1.wait()

      zero = jnp.zeros((LANES,), jnp.float32)

      def red(l, accs):
        a0, a1, a2, a3 = accs
        a0 = a0 + rows_v[l, pl.ds(0 * LANES, LANES)]
        a1 = a1 + rows_v[l, pl.ds(1 * LANES, LANES)]
        a2 = a2 + rows_v[l, pl.ds(2 * LANES, LANES)]
        a3 = a3 + rows_v[l, pl.ds(3 * LANES, LANES)]
        return (a0, a1, a2, a3)

      a0, a1, a2, a3 = lax.fori_loop(0, L, red, (zero, zero, zero, zero))

      grp = (b // LANES) * LANES
      lv = len_v[pl.ds(grp, LANES)]
      lenb = lax.gather(
          lv, jnp.full((LANES, 1), b - grp, jnp.int32),
          lax.GatherDimensionNumbers(
              offset_dims=(), collapsed_slice_dims=(0,), start_index_map=(0,)),
          (1,), mode=lax.GatherScatterMode.PROMISE_IN_BOUNDS)
      inv = 1.0 / lenb
      out_v[b, pl.ds(0 * LANES, LANES)] = a0 * inv
      out_v[b, pl.ds(1 * LANES, LANES)] = a1 * inv
      out_v[b, pl.ds(2 * LANES, LANES)] = a2 * inv
      out_v[b, pl.ds(3 * LANES, LANES)] = a3 * inv
      return 0

    lax.fori_loop(0, BPW, item_body, 0)

    pltpu.sync_copy(out_v, out_hbm.at[pl.ds(base, BPW)])

  return k


_sc_kernel = _build_sc_kernel()


def kernel(query, query_length, W):
  q = query.reshape(B, NCHUNK, CH)
  lens = query_length.astype(jnp.float32)
  return _sc_kernel(q, lens, W)


# speedup vs baseline: 1.2406x; 1.2406x over previous
import functools

import jax
import jax.numpy as jnp
from jax import lax
from jax.experimental import pallas as pl
from jax.experimental.pallas import tpu as pltpu
from jax.experimental.pallas import tpu_sc as plsc

VOCAB = 1000000
DIM = 64
B = 4096
L = 200

NC = 2
NS = 16
NW = NC * NS
BPW = B // NW
NCHUNK = 2
CH = L // NCHUNK
LANES = 16
NBUF = 4
UNROLL = 8


def _build_sc_kernel():
  mesh = plsc.VectorSubcoreMesh(core_axis_name="c", subcore_axis_name="s")

  @functools.partial(
      pl.kernel,
      mesh=mesh,
      compiler_params=pltpu.CompilerParams(use_tc_tiling_on_sc=False),
      out_type=jax.ShapeDtypeStruct((B, DIM), jnp.float32),
      scratch_types=[
          pltpu.VMEM((BPW, NCHUNK, CH), jnp.int32),
          pltpu.VMEM((BPW,), jnp.float32),
          pltpu.VMEM((NBUF, L, DIM), jnp.float32),
          pltpu.VMEM((BPW, DIM), jnp.float32),
          pltpu.SemaphoreType.DMA,
          pltpu.SemaphoreType.DMA,
          pltpu.SemaphoreType.DMA,
          pltpu.SemaphoreType.DMA,
      ],
  )
  def k(q_hbm, len_hbm, w_hbm, out_hbm, idx_v, len_v, rows_v, out_v, *sems):
    wid = lax.axis_index("s") * NC + lax.axis_index("c")
    base = wid * BPW

    pltpu.sync_copy(q_hbm.at[pl.ds(base, BPW)], idx_v)
    pltpu.sync_copy(len_hbm.at[pl.ds(base, BPW)], len_v)

    def start_gather(b, j):
      pltpu.async_copy(
          w_hbm.at[idx_v.at[b, 0]], rows_v.at[j, pl.ds(0, CH)], sems[j])
      pltpu.async_copy(
          w_hbm.at[idx_v.at[b, 1]], rows_v.at[j, pl.ds(CH, CH)], sems[j])

    def wait_gather(j):
      pltpu.make_async_copy(
          w_hbm.at[pl.ds(0, L)], rows_v.at[j], sems[j]).wait()

    for j in range(NBUF):
      start_gather(j, j)

    def group_body(g, _):
      for j in range(NBUF):
        b = g * NBUF + j
        wait_gather(j)

        zero = jnp.zeros((LANES,), jnp.float32)

        def red(i, accs):
          a0, a1, a2, a3 = accs
          l0 = i * UNROLL
          for r in range(UNROLL):
            a0 = a0 + rows_v[j, l0 + r, pl.ds(0 * LANES, LANES)]
            a1 = a1 + rows_v[j, l0 + r, pl.ds(1 * LANES, LANES)]
            a2 = a2 + rows_v[j, l0 + r, pl.ds(2 * LANES, LANES)]
            a3 = a3 + rows_v[j, l0 + r, pl.ds(3 * LANES, LANES)]
          return (a0, a1, a2, a3)

        a0, a1, a2, a3 = lax.fori_loop(
            0, L // UNROLL, red, (zero, zero, zero, zero))

        start_gather(jnp.minimum(b + NBUF, BPW - 1), j)

        grp = (b // LANES) * LANES
        lv = len_v[pl.ds(grp, LANES)]
        lenb = lax.gather(
            lv, jnp.full((LANES, 1), b - grp, jnp.int32),
            lax.GatherDimensionNumbers(
                offset_dims=(), collapsed_slice_dims=(0,),
                start_index_map=(0,)),
            (1,), mode=lax.GatherScatterMode.PROMISE_IN_BOUNDS)
        inv = 1.0 / lenb
        out_v[b, pl.ds(0 * LANES, LANES)] = a0 * inv
        out_v[b, pl.ds(1 * LANES, LANES)] = a1 * inv
        out_v[b, pl.ds(2 * LANES, LANES)] = a2 * inv
        out_v[b, pl.ds(3 * LANES, LANES)] = a3 * inv
      return 0

    lax.fori_loop(0, BPW // NBUF, group_body, 0)

    for j in range(NBUF):
      wait_gather(j)

    pltpu.sync_copy(out_v, out_hbm.at[pl.ds(base, BPW)])

  return k


_sc_kernel = _build_sc_kernel()


def kernel(query, query_length, W):
  q = query.reshape(B, NCHUNK, CH)
  lens = query_length.astype(jnp.float32)
  return _sc_kernel(q, lens, W)
